# Initial kernel scaffold; baseline (speedup 1.0000x reference)
import jax
import jax.numpy as jnp
from jax.experimental import pallas as pl


def kernel(indices, table0, table1, table2, table3, W0, W1, W2, W3):
    raise NotImplementedError("write your pallas kernel here")


# speedup vs baseline: 1.1447x; 1.1447x over previous
import functools

import jax
import jax.numpy as jnp
from jax import lax
from jax.experimental import pallas as pl
from jax.experimental.pallas import tpu as pltpu
from jax.experimental.pallas import tpu_sc as plsc

CUT = (0, 20_000, 100_000, 400_000, 1_000_000)
D = 128

B = 4096 * 20
NC, NS, L = 2, 16, 16
NW = NC * NS
TOK_PER_W = B // NW
CH = 128
NCH = TOK_PER_W // CH


def _sc_gather(idx_flat, t0, t1, t2v, t3v):
    mesh = plsc.VectorSubcoreMesh(core_axis_name="c", subcore_axis_name="s")

    @functools.partial(
        pl.kernel,
        mesh=mesh,
        compiler_params=pltpu.CompilerParams(use_tc_tiling_on_sc=False),
        out_type=(
            jax.ShapeDtypeStruct((B, 128), jnp.float32),
            jax.ShapeDtypeStruct((B, 32), jnp.float32),
            jax.ShapeDtypeStruct((B, 16), jnp.float32),
            jax.ShapeDtypeStruct((B, 16), jnp.float32),
        ),
        scratch_types=(
            pltpu.VMEM((CH,), jnp.int32),
            pltpu.VMEM((CH,), jnp.int32),
            pltpu.VMEM((CH,), jnp.int32),
            pltpu.VMEM((CH,), jnp.int32),
            pltpu.VMEM((CH,), jnp.int32),
            pltpu.VMEM((CH, 128), jnp.float32),
            pltpu.VMEM((CH, 32), jnp.float32),
            pltpu.VMEM((CH, 16), jnp.float32),
            pltpu.VMEM((CH, 16), jnp.float32),
            pltpu.SemaphoreType.DMA,
            pltpu.SemaphoreType.DMA,
            pltpu.SemaphoreType.DMA,
            pltpu.SemaphoreType.DMA,
        ),
    )
    def k(idx_hbm, t0_hbm, t1_hbm, t2_hbm, t3_hbm,
          g0_hbm, g1_hbm, g2_hbm, g3_hbm,
          idx_v, r0_v, r1_v, r2_v, r3_v,
          b0, b1, b2, b3,
          sem0, sem1, sem2, sem3):
        wid = lax.axis_index("s") * NC + lax.axis_index("c")
        tbase = wid * TOK_PER_W

        def body(s, carry):
            base = tbase + s * CH
            pltpu.sync_copy(idx_hbm.at[pl.ds(base, CH)], idx_v)
            for g in range(CH // L):
                v = idx_v[pl.ds(g * L, L)]
                r0_v[pl.ds(g * L, L)] = jnp.clip(v, 0, CUT[1] - 1)
                r1_v[pl.ds(g * L, L)] = jnp.clip(v - CUT[1], 0, CUT[2] - CUT[1] - 1)
                r2 = jnp.clip(v - CUT[2], 0, CUT[3] - CUT[2] - 1)
                r2_v[pl.ds(g * L, L)] = lax.shift_right_logical(r2, 1)
                r3 = jnp.clip(v - CUT[3], 0, CUT[4] - CUT[3] - 1)
                r3_v[pl.ds(g * L, L)] = lax.shift_right_logical(r3, 3)
            cp0 = pltpu.async_copy(t0_hbm.at[r0_v], b0, sem0)
            cp1 = pltpu.async_copy(t1_hbm.at[r1_v], b1, sem1)
            cp2 = pltpu.async_copy(t2_hbm.at[r2_v], b2, sem2)
            cp3 = pltpu.async_copy(t3_hbm.at[r3_v], b3, sem3)
            cp0.wait()
            cp1.wait()
            cp2.wait()
            cp3.wait()
            pltpu.sync_copy(b0, g0_hbm.at[pl.ds(base, CH)])
            pltpu.sync_copy(b1, g1_hbm.at[pl.ds(base, CH)])
            pltpu.sync_copy(b2, g2_hbm.at[pl.ds(base, CH)])
            pltpu.sync_copy(b3, g3_hbm.at[pl.ds(base, CH)])
            return carry

        lax.fori_loop(0, NCH, body, 0)

    return k(idx_flat, t0, t1, t2v, t3v)


def _tc_combine(idx_col, g0, g1, g2, g3, w0t, w1t, w2t, w3t):
    BT = 512
    NB = B // BT

    def body(idx_ref, g0_ref, g1_ref, g2_ref, g3_ref,
             w0_ref, w1_ref, w2_ref, w3_ref, out_ref):
        f32 = jnp.float32
        idx = idx_ref[...]
        m1 = idx >= CUT[1]
        m2 = idx >= CUT[2]
        m3 = idx >= CUT[3]
        k0 = jnp.logical_not(m1).astype(f32)
        k1 = jnp.logical_and(m1, jnp.logical_not(m2)).astype(f32)
        k2 = jnp.logical_and(m2, jnp.logical_not(m3)).astype(f32)
        k3 = m3.astype(f32)

        x0 = g0_ref[...] * k0
        x1 = g1_ref[...] * k1

        r2 = jnp.clip(idx - CUT[2], 0, CUT[3] - CUT[2] - 1)
        g2v = g2_ref[...]
        x2 = jnp.where((r2 & 1) == 1, g2v[:, 8:16], g2v[:, 0:8]) * k2

        r3 = jnp.clip(idx - CUT[3], 0, CUT[4] - CUT[3] - 1)
        p = (r3 & 7) * 2
        iota = lax.broadcasted_iota(jnp.int32, (BT, 16), 1)
        g3v = g3_ref[...]
        a = jnp.sum(jnp.where(iota == p, g3v, 0.0), axis=1, keepdims=True)
        b = jnp.sum(jnp.where(iota == p + 1, g3v, 0.0), axis=1, keepdims=True)

        acc = jnp.dot(x0, w0_ref[...], preferred_element_type=f32)
        acc = acc + jnp.dot(x1, w1_ref[...], preferred_element_type=f32)
        acc = acc + jnp.dot(x2, w2_ref[...], preferred_element_type=f32)
        acc = acc + (a * w3_ref[0:1, :] + b * w3_ref[1:2, :]) * k3
        out_ref[...] = acc

    return pl.pallas_call(
        body,
        grid=(NB,),
        in_specs=[
            pl.BlockSpec((BT, 1), lambda i: (i, 0)),
            pl.BlockSpec((BT, 128), lambda i: (i, 0)),
            pl.BlockSpec((BT, 32), lambda i: (i, 0)),
            pl.BlockSpec((BT, 16), lambda i: (i, 0)),
            pl.BlockSpec((BT, 16), lambda i: (i, 0)),
            pl.BlockSpec((128, 128), lambda i: (0, 0)),
            pl.BlockSpec((32, 128), lambda i: (0, 0)),
            pl.BlockSpec((8, 128), lambda i: (0, 0)),
            pl.BlockSpec((2, 128), lambda i: (0, 0)),
        ],
        out_specs=pl.BlockSpec((BT, 128), lambda i: (i, 0)),
        out_shape=jax.ShapeDtypeStruct((B, 128), jnp.float32),
    )(idx_col, g0, g1, g2, g3, w0t, w1t, w2t, w3t)


def kernel(indices, table0, table1, table2, table3, W0, W1, W2, W3):
    idx_flat = indices.reshape(B)
    t2v = table2.reshape(-1, 16)
    t3v = table3.reshape(-1, 16)
    g0, g1, g2, g3 = _sc_gather(idx_flat, table0, table1, t2v, t3v)
    out = _tc_combine(idx_flat.reshape(B, 1), g0, g1, g2, g3,
                      W0.T, W1.T, W2.T, W3.T)
    return out.reshape(indices.shape + (D,))


# speedup vs baseline: 5.3188x; 4.6464x over previous
import functools

import jax
import jax.numpy as jnp
from jax import lax
from jax.experimental import pallas as pl
from jax.experimental.pallas import tpu as pltpu
from jax.experimental.pallas import tpu_sc as plsc

CUT = (0, 20_000, 100_000, 400_000, 1_000_000)
D = 128

B = 4096 * 20
NC, NS, L = 2, 16, 16
NW = NC * NS
TOK_PER_W = B // NW
CH = 256
NCH = TOK_PER_W // CH
CAP = CH + 16
TB = 4

CLUSTERS = ((0, 128, 128), (128, 32, 32), (160, 8, 16), (168, 2, 16))


def _sc_kernel(idx_flat, t0, t1, t2v, t3v, wcat):
    mesh = plsc.VectorSubcoreMesh(core_axis_name="c", subcore_axis_name="s")

    @functools.partial(
        pl.kernel,
        mesh=mesh,
        compiler_params=pltpu.CompilerParams(
            use_tc_tiling_on_sc=False, needs_layout_passes=False),
        out_type=jax.ShapeDtypeStruct((B, 128), jnp.float32),
        scratch_types=(
            pltpu.VMEM((CAP,), jnp.int32),
            tuple(pltpu.VMEM((CAP,), jnp.int32) for _ in range(4)),
            tuple(pltpu.VMEM((CAP,), jnp.int32) for _ in range(4)),
            pltpu.VMEM((CAP, 128), jnp.float32),
            pltpu.VMEM((CAP, 32), jnp.float32),
            pltpu.VMEM((CAP, 16), jnp.float32),
            pltpu.VMEM((CAP, 16), jnp.float32),
            pltpu.VMEM((CAP, 128), jnp.float32),
            pltpu.VMEM((176, 128), jnp.float32),
            pltpu.SemaphoreType.DMA,
            pltpu.SemaphoreType.DMA,
            pltpu.SemaphoreType.DMA,
            pltpu.SemaphoreType.DMA,
        ),
    )
    def k(idx_hbm, t0_hbm, t1_hbm, t2_hbm, t3_hbm, w_hbm, out_hbm,
          idx_v, rel_v, slot_v, b0, b1, b2, b3, oc, wt,
          sem0, sem1, sem2, sem3):
        tables = (t0_hbm, t1_hbm, t2_hbm, t3_hbm)
        bufs = (b0, b1, b2, b3)
        sems = (sem0, sem1, sem2, sem3)

        wid = lax.axis_index("s") * NC + lax.axis_index("c")
        tbase = wid * TOK_PER_W
        pltpu.sync_copy(w_hbm, wt)

        lane = lax.iota(jnp.int32, L)

        def chunk_body(s, carry0):
            base = tbase + s * CH
            pltpu.sync_copy(idx_hbm.at[pl.ds(base, CH)], idx_v.at[pl.ds(0, CH)])

            def cgroup(g, cnts):
                v = idx_v[pl.ds(g * L, L)]
                slot = lane + g * L
                one = jnp.int32(1)
                zero = jnp.int32(0)
                c = (jnp.where(v >= CUT[1], one, zero)
                     + jnp.where(v >= CUT[2], one, zero)
                     + jnp.where(v >= CUT[3], one, zero))
                rows = (
                    jnp.clip(v, 0, CUT[1] - 1),
                    jnp.clip(v - CUT[1], 0, CUT[2] - CUT[1] - 1),
                    lax.shift_right_logical(
                        jnp.clip(v - CUT[2], 0, CUT[3] - CUT[2] - 1), 1),
                    lax.shift_right_logical(
                        jnp.clip(v - CUT[3], 0, CUT[4] - CUT[3] - 1), 3),
                )
                new = []
                for cc in range(4):
                    m = c == cc
                    cnt = cnts[cc]
                    cum = jnp.cumsum(jnp.where(m, one, zero))
                    pos = cnt + cum - 1
                    plsc.store_scatter(rel_v[cc], [pos], rows[cc], mask=m)
                    plsc.store_scatter(slot_v[cc], [pos], slot, mask=m)
                    new.append(cnt + cum[L - 1])
                return tuple(new)

            cnts = lax.fori_loop(0, CH // L, cgroup,
                                 (jnp.int32(0),) * 4)

            ngs = []
            for cc in range(4):
                tail = cnts[cc] + lane
                plsc.store_scatter(rel_v[cc], [tail],
                                   jnp.zeros((L,), jnp.int32))
                plsc.store_scatter(slot_v[cc], [tail],
                                   jnp.full((L,), CH, jnp.int32))
                ngs.append(lax.shift_right_logical(cnts[cc] + (L - 1), 4))

            for cc in range(4):
                tbl, buf, sem = tables[cc], bufs[cc], sems[cc]

                def fire(g, carry, tbl=tbl, buf=buf, sem=sem, cc=cc):
                    pltpu.async_copy(
                        tbl.at[rel_v[cc].at[pl.ds(g * L, L)]],
                        buf.at[pl.ds(g * L, L)], sem)
                    return carry

                lax.fori_loop(0, ngs[cc], fire, 0)
            for cc in range(4):
                tbl, buf, sem = tables[cc], bufs[cc], sems[cc]

                def drain(g, carry, tbl=tbl, buf=buf, sem=sem, cc=cc):
                    pltpu.make_async_copy(
                        tbl.at[rel_v[cc].at[pl.ds(0, L)]],
                        buf.at[pl.ds(0, L)], sem).wait()
                    return carry

                lax.fori_loop(0, ngs[cc], drain, 0)

            dnums = lax.GatherDimensionNumbers(
                offset_dims=(), collapsed_slice_dims=(0,),
                start_index_map=(0,))

            def vgather(vec, idxvec):
                return lax.gather(
                    vec, idxvec[:, None], dnums, (1,),
                    mode=lax.GatherScatterMode.PROMISE_IN_BOUNDS)

            def splat(vec, lane_const):
                return vgather(vec, jnp.full((L,), lane_const, jnp.int32))

            for cc, (roff, d, bw) in enumerate(CLUSTERS):
                buf = bufs[cc]

                def grp_body(t, carry, buf=buf, cc=cc, roff=roff, d=d, bw=bw):
                    slot16 = jnp.clip(slot_v[cc][pl.ds(t * L, L)], 0, CH)
                    tok16 = plsc.load_gather(idx_v, [slot16])
                    if cc == 2:
                        r = jnp.clip(tok16 - CUT[2], 0, CUT[3] - CUT[2] - 1)
                        sub16 = (r & 1) * 8
                    elif cc == 3:
                        r = jnp.clip(tok16 - CUT[3], 0, CUT[4] - CUT[3] - 1)
                        sub16 = (r & 7) * 2
                    else:
                        sub16 = None

                    zero8 = tuple(jnp.zeros((16,), jnp.float32)
                                  for _ in range(8))
                    for ub in range(L // TB):
                        us = tuple(ub * TB + u for u in range(TB))

                        if d > 16:
                            nkb = d // L

                            def kblock(kb, accs, buf=buf, roff=roff, us=us,
                                       bw=bw):
                                xr = [buf[t * L + u, pl.ds(kb * L, L)]
                                      for u in us]
                                for kk in range(L):
                                    wrow = [wt[roff + kb * L + kk,
                                               pl.ds(16 * v, 16)]
                                            for v in range(8)]
                                    accs = tuple(
                                        tuple(accs[i][v]
                                              + splat(xr[i], kk) * wrow[v]
                                              for v in range(8))
                                        for i in range(TB))
                                return accs

                            accs = lax.fori_loop(0, nkb, kblock,
                                                 (zero8,) * TB)
                        else:
                            xr = []
                            for u in us:
                                raw = buf[t * L + u, pl.ds(0, L)]
                                xr.append(vgather(
                                    raw, (splat(sub16, u) + lane) & (L - 1)))
                            accs = (zero8,) * TB
                            for kk in range(d):
                                wrow = [wt[roff + kk, pl.ds(16 * v, 16)]
                                        for v in range(8)]
                                accs = tuple(
                                    tuple(accs[i][v]
                                          + splat(xr[i], kk) * wrow[v]
                                          for v in range(8))
                                    for i in range(TB))

                        for i, u in enumerate(us):
                            su = slot16[u]
                            for v in range(8):
                                oc[su, pl.ds(16 * v, 16)] = accs[i][v]
                    return carry

                lax.fori_loop(0, ngs[cc], grp_body, 0)

            pltpu.sync_copy(oc.at[pl.ds(0, CH)], out_hbm.at[pl.ds(base, CH)])
            return carry0

        lax.fori_loop(0, NCH, chunk_body, 0)

    return k(idx_flat, t0, t1, t2v, t3v, wcat)


def kernel(indices, table0, table1, table2, table3, W0, W1, W2, W3):
    idx_flat = indices.reshape(B)
    t2v = table2.reshape(-1, 16)
    t3v = table3.reshape(-1, 16)
    wcat = jnp.concatenate(
        [W0.T, W1.T, W2.T, W3.T, jnp.zeros((6, 128), jnp.float32)], axis=0)
    out = _sc_kernel(idx_flat, table0, table1, t2v, t3v, wcat)
    return out.reshape(indices.shape + (D,))
